# baseline (speedup 1.0000x reference)
import functools

import jax
import jax.numpy as jnp
from jax import lax
from jax.experimental import pallas as pl
from jax.experimental.pallas import tpu as pltpu
from jax.experimental.pallas import tpu_sc as plsc

N = 10000
E = 320000
D = 128
NC, NS = 2, 16
NW = NC * NS
CH = 128
TOTCH = 2560
NCH = TOTCH // NW
K0, K1 = 80, 80
GROUP = 40
EP = TOTCH * CH
NP = 10112
STRIPE = NP // NS
BR = 1000

_mesh = plsc.VectorSubcoreMesh(core_axis_name="c", subcore_axis_name="s",
                               num_cores=NC, num_subcores=NS)


def _sc_scatter_body(z_hbm, src_hbm, dst_hbm, zeros_hbm, out_hbm,
                     src_v, dst_v, rows_v, acc, sem0, sem1, ssem0, ssem1):
    c = lax.axis_index("c")
    s = lax.axis_index("s")
    pltpu.sync_copy(zeros_hbm, acc.at[pl.ds(s * STRIPE, STRIPE)])
    plsc.subcore_barrier()

    cbase = jnp.where(c == 0, s * K0, NS * K0 + s * K1)
    ngroups = jnp.where(c == 0, K0 // GROUP, K1 // GROUP)

    def group(g, carry):
        gbase = pl.multiple_of(cbase + g * GROUP, 8)
        pltpu.sync_copy(src_hbm.at[pl.ds(gbase, GROUP)], src_v)
        pltpu.sync_copy(dst_hbm.at[pl.ds(gbase, GROUP)], dst_v)
        pltpu.async_copy(z_hbm.at[src_v.at[0]], rows_v.at[0], sem0)
        pltpu.async_copy(z_hbm.at[src_v.at[1]], rows_v.at[1], sem1)

        def pair(j, carry2):
            i0 = 2 * j
            i1 = i0 + 1
            pltpu.make_async_copy(z_hbm.at[src_v.at[i0]], rows_v.at[0],
                                  sem0).wait()
            pltpu.async_copy(rows_v.at[0], acc.at[dst_v.at[i0]], ssem0,
                             add=True)
            pltpu.make_async_copy(z_hbm.at[src_v.at[i1]], rows_v.at[1],
                                  sem1).wait()
            pltpu.async_copy(rows_v.at[1], acc.at[dst_v.at[i1]], ssem1,
                             add=True)
            pltpu.make_async_copy(rows_v.at[0], acc.at[dst_v.at[i0]],
                                  ssem0).wait()

            @pl.when(i0 + 2 < GROUP)
            def _():
                pltpu.async_copy(z_hbm.at[src_v.at[i0 + 2]], rows_v.at[0],
                                 sem0)

            pltpu.make_async_copy(rows_v.at[1], acc.at[dst_v.at[i1]],
                                  ssem1).wait()

            @pl.when(i1 + 2 < GROUP)
            def _():
                pltpu.async_copy(z_hbm.at[src_v.at[i1 + 2]], rows_v.at[1],
                                 sem1)

            return carry2

        lax.fori_loop(0, GROUP // 2, pair, 0)
        return carry

    lax.fori_loop(0, ngroups, group, 0)
    plsc.subcore_barrier()
    pltpu.sync_copy(acc.at[pl.ds(s * STRIPE, STRIPE)],
                    out_hbm.at[c, pl.ds(s * STRIPE, STRIPE)])


def _build_sc_scatter(interpret=False):
    return pl.kernel(
        _sc_scatter_body,
        out_type=jax.ShapeDtypeStruct((NC, NP, D), jnp.float32),
        mesh=_mesh,
        scratch_types=[
            pltpu.VMEM((GROUP, CH), jnp.int32),
            pltpu.VMEM((GROUP, CH), jnp.int32),
            pltpu.VMEM((2, CH, D), jnp.float32),
            pltpu.VMEM_SHARED((NP, D), jnp.float32),
            pltpu.SemaphoreType.DMA,
            pltpu.SemaphoreType.DMA,
            pltpu.SemaphoreType.DMA,
            pltpu.SemaphoreType.DMA,
        ],
        interpret=interpret,
    )


_sc_scatter = _build_sc_scatter()


def _sc_degree_body(dst_hbm, ones_hbm, zeros_hbm, out_hbm, dst_v, ones_v, acc):
    c = lax.axis_index("c")
    s = lax.axis_index("s")
    wid = s * NC + c
    pltpu.sync_copy(zeros_hbm, acc.at[pl.ds(s * STRIPE, STRIPE)])
    pltpu.sync_copy(ones_hbm, ones_v)
    pltpu.sync_copy(dst_hbm.at[pl.ds(wid * NCH, NCH)], dst_v)
    plsc.subcore_barrier()

    def chunk(i, carry):
        pltpu.sync_copy(ones_v, acc.at[dst_v.at[i]], add=True)
        return carry

    lax.fori_loop(0, NCH, chunk, 0)
    plsc.subcore_barrier()
    pltpu.sync_copy(acc.at[pl.ds(s * STRIPE, STRIPE)],
                    out_hbm.at[c, pl.ds(s * STRIPE, STRIPE)])


def _build_sc_degree(interpret=False):
    return pl.kernel(
        _sc_degree_body,
        out_type=jax.ShapeDtypeStruct((NC, NP, D), jnp.float32),
        mesh=_mesh,
        scratch_types=[
            pltpu.VMEM((NCH, CH), jnp.int32),
            pltpu.VMEM((CH, D), jnp.float32),
            pltpu.VMEM_SHARED((NP, D), jnp.float32),
        ],
        interpret=interpret,
    )


_sc_degree = _build_sc_degree()


def _dot(a, b):
    return jnp.dot(a, b, preferred_element_type=jnp.float32)


def _tc_first_body(x_r, wl_r, wr_r, b_r, z_o, r_o):
    h = x_r[...]
    z_o[...] = _dot(h, wl_r[...])
    r_o[...] = _dot(h, wr_r[...]) + b_r[...]


def _tc_mid_body(aa_r, ab_r, da_r, db_r, rp_r, wl_r, wr_r, b_r, z_o, r_o):
    deg = da_r[0, :, 0:1] + db_r[0, :, 0:1]
    inv = 1.0 / jnp.maximum(deg, 1.0)
    h = (aa_r[0] + ab_r[0]) * inv + rp_r[...]
    h = jnp.where(h > 0, h, 0.1 * h)
    z_o[...] = _dot(h, wl_r[...])
    r_o[...] = _dot(h, wr_r[...]) + b_r[...]


def _tc_final_body(aa_r, ab_r, da_r, db_r, rp_r, out_o):
    deg = da_r[0, :, 0:1] + db_r[0, :, 0:1]
    inv = 1.0 / jnp.maximum(deg, 1.0)
    out_o[...] = (aa_r[0] + ab_r[0]) * inv + rp_r[...]


_row_spec = pl.BlockSpec((BR, D), lambda i: (i, 0))
_w_spec = pl.BlockSpec((D, D), lambda i: (0, 0))
_b_spec = pl.BlockSpec((1, D), lambda i: (0, 0))


def _acc_spec(core):
    return pl.BlockSpec((1, BR, D), lambda i: (core, i, 0))


def _deg_spec(core):
    return pl.BlockSpec((1, BR, D), lambda i: (core, i, 0))


_tc_first = pl.pallas_call(
    _tc_first_body,
    grid=(N // BR,),
    in_specs=[_row_spec, _w_spec, _w_spec, _b_spec],
    out_specs=[_row_spec, _row_spec],
    out_shape=[jax.ShapeDtypeStruct((N, D), jnp.float32)] * 2,
)

_tc_mid = pl.pallas_call(
    _tc_mid_body,
    grid=(N // BR,),
    in_specs=[_acc_spec(0), _acc_spec(1), _deg_spec(0), _deg_spec(1),
              _row_spec, _w_spec, _w_spec, _b_spec],
    out_specs=[_row_spec, _row_spec],
    out_shape=[jax.ShapeDtypeStruct((N, D), jnp.float32)] * 2,
)

_tc_final = pl.pallas_call(
    _tc_final_body,
    grid=(N // BR,),
    in_specs=[_acc_spec(0), _acc_spec(1), _deg_spec(0), _deg_spec(1), _row_spec],
    out_specs=_row_spec,
    out_shape=jax.ShapeDtypeStruct((N, D), jnp.float32),
)


def kernel(x, edge_index, Wl0, Wr0, b0, Wl1, Wr1, b1, Wl2, Wr2, b2,
           Wl3, Wr3, b3):
    src = edge_index[0].astype(jnp.int32)
    dst = edge_index[1].astype(jnp.int32)
    pad = EP - E
    trash = N + jnp.arange(pad, dtype=jnp.int32) % (NP - N)
    fake_src = jnp.arange(pad, dtype=jnp.int32) % N
    srcp = jnp.concatenate([src, fake_src]).reshape(TOTCH, CH)
    dstp = jnp.concatenate([dst, trash]).reshape(TOTCH, CH)
    zeros_l = jnp.zeros((STRIPE, D), jnp.float32)
    ones_d = jnp.ones((CH, D), jnp.float32)

    deg2 = _sc_degree(dstp, ones_d, zeros_l)
    z, r = _tc_first(x, Wl0, Wr0, b0.reshape(1, D))
    for wl, wr, b in ((Wl1, Wr1, b1), (Wl2, Wr2, b2), (Wl3, Wr3, b3)):
        acc2 = _sc_scatter(z, srcp, dstp, zeros_l)
        z, r = _tc_mid(acc2, acc2, deg2, deg2, r, wl, wr, b.reshape(1, D))
    acc2 = _sc_scatter(z, srcp, dstp, zeros_l)
    return _tc_final(acc2, acc2, deg2, deg2, r)


# speedup vs baseline: 1.2653x; 1.2653x over previous
import functools

import jax
import jax.numpy as jnp
from jax import lax
from jax.experimental import pallas as pl
from jax.experimental.pallas import tpu as pltpu
from jax.experimental.pallas import tpu_sc as plsc

N = 10000
E = 320000
D = 128
NC, NS = 2, 16
NW = NC * NS
CH = 128
TOTCH = 2560
NCH = TOTCH // NW
K0, K1 = 80, 80
GROUP = 40
EP = TOTCH * CH
NP = 10112
STRIPE = NP // NS
BR = 1000

_mesh = plsc.VectorSubcoreMesh(core_axis_name="c", subcore_axis_name="s",
                               num_cores=NC, num_subcores=NS)


def _sc_scatter_body(z_hbm, src_hbm, dst_hbm, zeros_hbm, out_hbm,
                     src_v, dst_v, rows_v, acc, sem0, sem1):
    c = lax.axis_index("c")
    s = lax.axis_index("s")
    pltpu.sync_copy(zeros_hbm, acc.at[pl.ds(s * STRIPE, STRIPE)])
    plsc.subcore_barrier()

    wid = s * NC + c
    pltpu.sync_copy(src_hbm.at[pl.ds(wid * NCH, NCH)], src_v)
    pltpu.async_copy(z_hbm.at[src_v.at[0]], rows_v.at[0], sem0)
    for p in range(NCH // GROUP):
        pltpu.sync_copy(
            dst_hbm.at[pl.ds(wid * NCH + p * GROUP, GROUP)], dst_v)

        def pair(j, carry, p=p):
            g0 = p * GROUP + 2 * j
            g1 = g0 + 1
            pltpu.async_copy(z_hbm.at[src_v.at[g1]], rows_v.at[1], sem1)
            pltpu.make_async_copy(z_hbm.at[src_v.at[g0]], rows_v.at[0],
                                  sem0).wait()
            pltpu.sync_copy(rows_v.at[0], acc.at[dst_v.at[2 * j]], add=True)

            @pl.when(g0 + 2 < NCH)
            def _():
                pltpu.async_copy(z_hbm.at[src_v.at[g0 + 2]], rows_v.at[0],
                                 sem0)

            pltpu.make_async_copy(z_hbm.at[src_v.at[g1]], rows_v.at[1],
                                  sem1).wait()
            pltpu.sync_copy(rows_v.at[1], acc.at[dst_v.at[2 * j + 1]],
                            add=True)
            return carry

        lax.fori_loop(0, GROUP // 2, pair, 0)
    plsc.subcore_barrier()
    pltpu.sync_copy(acc.at[pl.ds(s * STRIPE, STRIPE)],
                    out_hbm.at[c, pl.ds(s * STRIPE, STRIPE)])


def _build_sc_scatter(interpret=False):
    return pl.kernel(
        _sc_scatter_body,
        out_type=jax.ShapeDtypeStruct((NC, NP, D), jnp.float32),
        mesh=_mesh,
        scratch_types=[
            pltpu.VMEM((NCH, CH), jnp.int32),
            pltpu.VMEM((GROUP, CH), jnp.int32),
            pltpu.VMEM((2, CH, D), jnp.float32),
            pltpu.VMEM_SHARED((NP, D), jnp.float32),
            pltpu.SemaphoreType.DMA,
            pltpu.SemaphoreType.DMA,
        ],
        interpret=interpret,
    )


_sc_scatter = _build_sc_scatter()


def _sc_degree_body(dst_hbm, ones_hbm, zeros_hbm, out_hbm, dst_v, ones_v, acc):
    c = lax.axis_index("c")
    s = lax.axis_index("s")
    wid = s * NC + c
    pltpu.sync_copy(zeros_hbm, acc.at[pl.ds(s * STRIPE, STRIPE)])
    pltpu.sync_copy(ones_hbm, ones_v)
    pltpu.sync_copy(dst_hbm.at[pl.ds(wid * NCH, NCH)], dst_v)
    plsc.subcore_barrier()

    def chunk(i, carry):
        pltpu.sync_copy(ones_v, acc.at[dst_v.at[i]], add=True)
        return carry

    lax.fori_loop(0, NCH, chunk, 0)
    plsc.subcore_barrier()
    pltpu.sync_copy(acc.at[pl.ds(s * STRIPE, STRIPE)],
                    out_hbm.at[c, pl.ds(s * STRIPE, STRIPE)])


def _build_sc_degree(interpret=False):
    return pl.kernel(
        _sc_degree_body,
        out_type=jax.ShapeDtypeStruct((NC, NP, D), jnp.float32),
        mesh=_mesh,
        scratch_types=[
            pltpu.VMEM((NCH, CH), jnp.int32),
            pltpu.VMEM((CH, D), jnp.float32),
            pltpu.VMEM_SHARED((NP, D), jnp.float32),
        ],
        interpret=interpret,
    )


_sc_degree = _build_sc_degree()


def _dot(a, b):
    return jnp.dot(a, b, preferred_element_type=jnp.float32)


def _tc_first_body(x_r, wl_r, wr_r, b_r, z_o, r_o):
    h = x_r[...]
    z_o[...] = _dot(h, wl_r[...])
    r_o[...] = _dot(h, wr_r[...]) + b_r[...]


def _tc_mid_body(aa_r, ab_r, da_r, db_r, rp_r, wl_r, wr_r, b_r, z_o, r_o):
    deg = da_r[0, :, 0:1] + db_r[0, :, 0:1]
    inv = 1.0 / jnp.maximum(deg, 1.0)
    h = (aa_r[0] + ab_r[0]) * inv + rp_r[...]
    h = jnp.where(h > 0, h, 0.1 * h)
    z_o[...] = _dot(h, wl_r[...])
    r_o[...] = _dot(h, wr_r[...]) + b_r[...]


def _tc_final_body(aa_r, ab_r, da_r, db_r, rp_r, out_o):
    deg = da_r[0, :, 0:1] + db_r[0, :, 0:1]
    inv = 1.0 / jnp.maximum(deg, 1.0)
    out_o[...] = (aa_r[0] + ab_r[0]) * inv + rp_r[...]


_row_spec = pl.BlockSpec((BR, D), lambda i: (i, 0))
_w_spec = pl.BlockSpec((D, D), lambda i: (0, 0))
_b_spec = pl.BlockSpec((1, D), lambda i: (0, 0))


def _acc_spec(core):
    return pl.BlockSpec((1, BR, D), lambda i: (core, i, 0))


def _deg_spec(core):
    return pl.BlockSpec((1, BR, D), lambda i: (core, i, 0))


_tc_first = pl.pallas_call(
    _tc_first_body,
    grid=(N // BR,),
    in_specs=[_row_spec, _w_spec, _w_spec, _b_spec],
    out_specs=[_row_spec, _row_spec],
    out_shape=[jax.ShapeDtypeStruct((N, D), jnp.float32)] * 2,
)

_tc_mid = pl.pallas_call(
    _tc_mid_body,
    grid=(N // BR,),
    in_specs=[_acc_spec(0), _acc_spec(1), _deg_spec(0), _deg_spec(1),
              _row_spec, _w_spec, _w_spec, _b_spec],
    out_specs=[_row_spec, _row_spec],
    out_shape=[jax.ShapeDtypeStruct((N, D), jnp.float32)] * 2,
)

_tc_final = pl.pallas_call(
    _tc_final_body,
    grid=(N // BR,),
    in_specs=[_acc_spec(0), _acc_spec(1), _deg_spec(0), _deg_spec(1), _row_spec],
    out_specs=_row_spec,
    out_shape=jax.ShapeDtypeStruct((N, D), jnp.float32),
)


def kernel(x, edge_index, Wl0, Wr0, b0, Wl1, Wr1, b1, Wl2, Wr2, b2,
           Wl3, Wr3, b3):
    src = edge_index[0].astype(jnp.int32)
    dst = edge_index[1].astype(jnp.int32)
    pad = EP - E
    trash = N + jnp.arange(pad, dtype=jnp.int32) % (NP - N)
    fake_src = jnp.arange(pad, dtype=jnp.int32) % N
    srcp = jnp.concatenate([src, fake_src]).reshape(TOTCH, CH)
    dstp = jnp.concatenate([dst, trash]).reshape(TOTCH, CH)
    zeros_l = jnp.zeros((STRIPE, D), jnp.float32)
    ones_d = jnp.ones((CH, D), jnp.float32)

    deg2 = _sc_degree(dstp, ones_d, zeros_l)
    z, r = _tc_first(x, Wl0, Wr0, b0.reshape(1, D))
    for wl, wr, b in ((Wl1, Wr1, b1), (Wl2, Wr2, b2), (Wl3, Wr3, b3)):
        acc2 = _sc_scatter(z, srcp, dstp, zeros_l)
        z, r = _tc_mid(acc2, acc2, deg2, deg2, r, wl, wr, b.reshape(1, D))
    acc2 = _sc_scatter(z, srcp, dstp, zeros_l)
    return _tc_final(acc2, acc2, deg2, deg2, r)
